# baseline (speedup 1.0000x reference)
import functools

import jax
import jax.numpy as jnp
from jax import lax
from jax.experimental import pallas as pl
from jax.experimental.pallas import tpu as pltpu
from jax.experimental.pallas import tpu_sc as plsc

_D = 64
_SCALE = float(_D) ** 0.5
_NC = 2
_NS = 16
_NW = _NC * _NS
_LANES = 16
_NB = 4
_BLK = 128


@functools.partial(jax.jit, static_argnums=(2,))
def _lookup(x_t, table2, n_l):
  mesh = plsc.VectorSubcoreMesh(
      core_axis_name="c", subcore_axis_name="s", num_cores=_NC,
      num_subcores=_NS)

  @functools.partial(
      pl.kernel,
      mesh=mesh,
      out_type=jax.ShapeDtypeStruct((n_l, _NW * _BLK // 2, 2 * _D),
                                    jnp.float32),
      scratch_types=[
          pltpu.VMEM((n_l, _BLK), jnp.int32),
          [pltpu.VMEM((_BLK,), jnp.int32) for _ in range(_NB)],
          [pltpu.VMEM((_BLK, 2 * _D), jnp.float32) for _ in range(_NB)],
          [pltpu.VMEM((_BLK // 2, 2 * _D), jnp.float32) for _ in range(_NB)],
          [pltpu.SemaphoreType.DMA for _ in range(_NB)],
          [pltpu.SemaphoreType.DMA for _ in range(_NB)],
      ],
      compiler_params=pltpu.CompilerParams(
          use_tc_tiling_on_sc=False, needs_layout_passes=False),
  )
  def body(x_hbm, tab_hbm, out_hbm, idx_v, jrow, rows, outb, sem_g, sem_s):
    wid = lax.axis_index("s") * _NC + lax.axis_index("c")
    pltpu.sync_copy(x_hbm.at[:, pl.ds(wid * _BLK, _BLK)], idx_v)

    def start_gather(l, b):
      for k in range(_BLK // _LANES):
        sl = pl.ds(k * _LANES, _LANES)
        jrow[b][sl] = lax.shift_right_logical(idx_v[l, sl], 1)
      pltpu.make_async_copy(
          tab_hbm.at[jrow[b]], rows[b], sem_g[b]).start()

    def wait_gather(l, b):
      pltpu.make_async_copy(
          tab_hbm.at[jrow[b]], rows[b], sem_g[b]).wait()

    def scatter(l, b):
      return pltpu.make_async_copy(
          outb[b], out_hbm.at[l, pl.ds(wid * (_BLK // 2), _BLK // 2), :],
          sem_s[b])

    for b in range(_NB - 1):
      start_gather(b, b)

    def group_body(go, carry):
      for b in range(_NB):
        l = go * _NB + b
        wait_gather(l, b)

        @plsc.parallel_loop(0, _BLK // _LANES)
        def _(rb):
          hv = (idx_v[l, pl.ds(rb * _LANES, _LANES)] & 1) * _D
          for ri in range(_LANES):
            src0 = hv[ri]
            for k in range(_D // _LANES):
              v = rows[b][rb * _LANES + ri,
                          pl.ds(src0 + k * _LANES, _LANES)]
              outb[b][rb * (_LANES // 2) + ri // 2,
                      pl.ds((ri & 1) * _D + k * _LANES, _LANES)] = v * _SCALE

        scatter(l, b).start()

        nb = (b + _NB - 1) % _NB
        nl = l + _NB - 1

        @pl.when(nl < n_l)
        def _():
          @pl.when(l >= 1)
          def _():
            scatter(l - 1, nb).wait()
          start_gather(nl, nb)

      return carry

    lax.fori_loop(0, n_l // _NB, group_body, 0)
    for b in range(_NB):
      scatter(n_l - _NB + b, b).wait()

  return body(x_t, table2)


def kernel(x, table):
  b, l = x.shape
  x_t = jnp.transpose(x).astype(jnp.int32)
  table2 = jnp.reshape(table, (-1, 2 * _D))
  packed = _lookup(x_t, table2, l)
  rows_out = jnp.reshape(packed, (l, b, _D))
  return jnp.transpose(rows_out, (1, 0, 2))
